# baseline (speedup 1.0000x reference)
import functools

import jax
import jax.numpy as jnp
import numpy as np
from jax import lax
from jax.experimental import pallas as pl
from jax.experimental.pallas import tpu as pltpu
from jax.experimental.pallas import tpu_sc as plsc

TEMP = 0.1
B, N, D = 16, 4096, 64
TOT = B * N
G = 1024
NC, NS = 2, 16
NW = NC * NS
CH = TOT // NW
NVEC = CH // 16
NBATCH = CH // 128

_I32 = jnp.int32
_F32 = jnp.float32

def _wid():
    return lax.axis_index("s") * NC + lax.axis_index("c")


def _k1_body(groups_hbm, posw_hbm, prevw_hbm, cnt_hbm, lo_hbm,
             g_buf, posw_buf, prevw_buf, cnt_tbl, lo_tbl):
    wid = _wid()
    cbase = wid * CH
    pltpu.sync_copy(groups_hbm.at[pl.ds(cbase, CH)], g_buf)
    iota = lax.iota(_I32, 16)
    zeros16 = iota * 0
    false16 = iota < 0

    def init(i, _):
        cnt_tbl[pl.ds(i * 16, 16)] = zeros16
        lo_tbl[pl.ds(i * 16, 16)] = zeros16 - 1
        return 0

    lax.fori_loop(0, G // 16, init, 0)

    def step(v, _):
        off = v * 16
        g = g_buf[pl.ds(off, 16)]
        jvec = cbase + off + iota
        prior = zeros16
        prevj = zeros16
        found = false16
        anyf = false16
        for s in range(1, 16):
            gb = plsc.load_gather(g_buf, [off + jnp.maximum(iota - s, 0)])
            eqb = (gb == g) & (iota >= s)
            prevj = jnp.where(eqb & (~found), jvec - s, prevj)
            found = found | eqb
            prior = prior + eqb.astype(_I32)
            gf = plsc.load_gather(g_buf, [off + jnp.minimum(iota + s, 15)])
            anyf = anyf | ((gf == g) & (iota < 16 - s))
        cnt_cur = plsc.load_gather(cnt_tbl, [g])
        lo_cur = plsc.load_gather(lo_tbl, [g])
        posw = cnt_cur + prior
        prevv = jnp.where(found, prevj, lo_cur)
        islast = jnp.logical_not(anyf)
        plsc.store_scatter(cnt_tbl, [g], posw + 1, mask=islast)
        plsc.store_scatter(lo_tbl, [g], jvec, mask=islast)
        posw_buf[pl.ds(off, 16)] = posw
        prevw_buf[pl.ds(off, 16)] = prevv
        return 0

    lax.fori_loop(0, NVEC, step, 0)
    pltpu.sync_copy(posw_buf, posw_hbm.at[pl.ds(cbase, CH)])
    pltpu.sync_copy(prevw_buf, prevw_hbm.at[pl.ds(cbase, CH)])
    pltpu.sync_copy(cnt_tbl, cnt_hbm.at[wid])
    pltpu.sync_copy(lo_tbl, lo_hbm.at[wid])


_k1 = functools.partial(
    pl.kernel,
    out_type=(
        jax.ShapeDtypeStruct((TOT,), _I32),
        jax.ShapeDtypeStruct((TOT,), _I32),
        jax.ShapeDtypeStruct((NW, G), _I32),
        jax.ShapeDtypeStruct((NW, G), _I32),
    ),
    mesh=plsc.VectorSubcoreMesh(core_axis_name="c", subcore_axis_name="s"),
    compiler_params=pltpu.CompilerParams(needs_layout_passes=False),
    scratch_types=(
        pltpu.VMEM((CH,), _I32),
        pltpu.VMEM((CH,), _I32),
        pltpu.VMEM((CH,), _I32),
        pltpu.VMEM((G,), _I32),
        pltpu.VMEM((G,), _I32),
    ),
)(_k1_body)


def _k2_body(cnt_ref, lo_ref, base_ref, il_ref, hist_ref):
    cntf = cnt_ref[...].astype(_F32)
    lo = lo_ref[...]
    ri = lax.broadcasted_iota(_I32, (NW, NW), 0)
    ci = lax.broadcasted_iota(_I32, (NW, NW), 1)
    tril = (ci < ri).astype(_F32)
    pre = lax.dot(tril, cntf, preferred_element_type=_F32)
    hist = jnp.sum(cntf, axis=0, keepdims=True)
    gi = lax.broadcasted_iota(_I32, (G, G), 0)
    gj = lax.broadcasted_iota(_I32, (G, G), 1)
    sl = (gi < gj).astype(_F32)
    off = lax.dot(hist, sl, preferred_element_type=_F32)
    base_ref[...] = (off + pre).astype(_I32)
    m = lo
    for step in (1, 2, 4, 8, 16):
        shifted = jnp.concatenate(
            [jnp.full((step, G), -1, _I32), m[:-step]], axis=0)
        m = jnp.maximum(m, shifted)
    excl = jnp.concatenate([jnp.full((1, G), -1, _I32), m[:-1]], axis=0)
    gmax = m[-1:]
    il_ref[...] = jnp.where(excl >= 0, excl, gmax)
    hist_ref[...] = jnp.broadcast_to(hist.astype(_I32), (8, G))


def _k2(cnt, lo):
    return pl.pallas_call(
        _k2_body,
        out_shape=(
            jax.ShapeDtypeStruct((NW, G), _I32),
            jax.ShapeDtypeStruct((NW, G), _I32),
            jax.ShapeDtypeStruct((8, G), _I32),
        ),
    )(cnt, lo)


def _k3_body(emb_hbm, groups_hbm, neg_hbm, posw_hbm, prevw_hbm,
             base_hbm, il_hbm, hist_hbm,
             s_hbm, w_hbm,
             g_buf, posw_b, prevw_b, base_t, il_t, hist_t,
             rank2d, part2d, negid2d, negg2d, s_buf, w_buf,
             anchor_v, pos_v, neg_v, sem):
    wid = _wid()
    cbase = wid * CH
    pltpu.sync_copy(groups_hbm.at[pl.ds(cbase, CH)], g_buf)
    pltpu.sync_copy(posw_hbm.at[pl.ds(cbase, CH)], posw_b)
    pltpu.sync_copy(prevw_hbm.at[pl.ds(cbase, CH)], prevw_b)
    pltpu.sync_copy(base_hbm.at[wid], base_t)
    pltpu.sync_copy(il_hbm.at[wid], il_t)
    pltpu.sync_copy(hist_hbm.at[0], hist_t)
    iota = lax.iota(_I32, 16)
    zeros16 = iota * 0
    fzeros16 = iota * 0.0

    def phase_a(v, _):
        off = v * 16
        r = v // 8
        col = (v % 8) * 16
        g = g_buf[pl.ds(off, 16)]
        pw = posw_b[pl.ds(off, 16)]
        pv = prevw_b[pl.ds(off, 16)]
        rank = plsc.load_gather(base_t, [g]) + pw
        part = jnp.where(pv >= 0, pv, plsc.load_gather(il_t, [g]))
        rank2d[r, pl.ds(col, 16)] = rank
        part2d[r, pl.ds(col, 16)] = part
        return 0

    lax.fori_loop(0, NVEC, phase_a, 0)

    handles = [pltpu.async_copy(neg_hbm.at[rank2d.at[r]], negid2d.at[r], sem)
               for r in range(NBATCH)]
    for h in handles:
        h.wait()
    handles = [pltpu.async_copy(groups_hbm.at[negid2d.at[r]], negg2d.at[r], sem)
               for r in range(NBATCH)]
    for h in handles:
        h.wait()

    def phase_c(r, _):
        cp_p = pltpu.async_copy(emb_hbm.at[part2d.at[r]], pos_v, sem)
        cp_n = pltpu.async_copy(emb_hbm.at[negid2d.at[r]], neg_v, sem)
        cp_a = pltpu.async_copy(emb_hbm.at[pl.ds(cbase + r * 128, 128)],
                                anchor_v, sem)
        cp_p.wait()
        cp_n.wait()
        cp_a.wait()

        def vec(n, _):
            rowidx = n * 16 + iota
            acc_p = fzeros16
            acc_n = fzeros16
            for e in range(D):
                ecol = zeros16 + e
                a = plsc.load_gather(anchor_v, [rowidx, ecol])
                p = plsc.load_gather(pos_v, [rowidx, ecol])
                q = plsc.load_gather(neg_v, [rowidx, ecol])
                acc_p = acc_p + a * p
                acc_n = acc_n + a * q
            off = r * 128 + n * 16
            g = g_buf[pl.ds(off, 16)]
            negg = negg2d[r, pl.ds(n * 16, 16)]
            size = plsc.load_gather(hist_t, [g])
            w = ((size >= 2) & (negg != g)).astype(_F32)
            s_buf[pl.ds(off, 16)] = acc_n - acc_p
            w_buf[pl.ds(off, 16)] = w
            return 0

        lax.fori_loop(0, 8, vec, 0)
        return 0

    lax.fori_loop(0, NBATCH, phase_c, 0)
    pltpu.sync_copy(s_buf, s_hbm.at[pl.ds(cbase, CH)])
    pltpu.sync_copy(w_buf, w_hbm.at[pl.ds(cbase, CH)])


_k3 = functools.partial(
    pl.kernel,
    out_type=(
        jax.ShapeDtypeStruct((TOT,), _F32),
        jax.ShapeDtypeStruct((TOT,), _F32),
    ),
    mesh=plsc.VectorSubcoreMesh(core_axis_name="c", subcore_axis_name="s"),
    compiler_params=pltpu.CompilerParams(needs_layout_passes=False),
    scratch_types=(
        pltpu.VMEM((CH,), _I32),
        pltpu.VMEM((CH,), _I32),
        pltpu.VMEM((CH,), _I32),
        pltpu.VMEM((G,), _I32),
        pltpu.VMEM((G,), _I32),
        pltpu.VMEM((G,), _I32),
        pltpu.VMEM((NBATCH, 128), _I32),
        pltpu.VMEM((NBATCH, 128), _I32),
        pltpu.VMEM((NBATCH, 128), _I32),
        pltpu.VMEM((NBATCH, 128), _I32),
        pltpu.VMEM((CH,), _F32),
        pltpu.VMEM((CH,), _F32),
        pltpu.VMEM((128, 128), _F32),
        pltpu.VMEM((128, 128), _F32),
        pltpu.VMEM((128, 128), _F32),
        pltpu.SemaphoreType.DMA,
    ),
)(_k3_body)


def _k4_body(s_ref, w_ref, out_ref):
    sc = s_ref[...] / TEMP
    sp = jnp.maximum(sc, 0.0) + jnp.log(1.0 + jnp.exp(-jnp.abs(sc)))
    w = w_ref[...]
    num = jnp.sum(sp * w)
    den = jnp.sum(w)
    out_ref[...] = jnp.full((8, 128), num / jnp.maximum(den, 1.0), _F32)


def _k4(s2, w2):
    return pl.pallas_call(
        _k4_body,
        out_shape=jax.ShapeDtypeStruct((8, 128), _F32),
    )(s2, w2)


def kernel(embeddings, groups):
    emb_flat = embeddings.reshape(TOT, D)
    emb_pad = jnp.pad(emb_flat, ((0, 0), (0, 128 - D)))
    g_flat = groups.reshape(TOT).astype(_I32)
    neg_const = jax.random.randint(jax.random.key(123), (TOT,), 0, TOT,
                                   dtype=_I32)
    posw, prevw, cnt, lo = _k1(g_flat)
    base, il, hist = _k2(cnt, lo)
    sdiff, w = _k3(emb_pad, g_flat, neg_const, posw, prevw, base, il, hist)
    nd = _k4(sdiff.reshape(512, 128), w.reshape(512, 128))
    return nd[0, 0]


# speedup vs baseline: 1.7703x; 1.7703x over previous
import functools

import jax
import jax.numpy as jnp
import numpy as np
from jax import lax
from jax.experimental import pallas as pl
from jax.experimental.pallas import tpu as pltpu
from jax.experimental.pallas import tpu_sc as plsc

TEMP = 0.1
B, N, D = 16, 4096, 64
TOT = B * N
G = 1024
NC, NS = 2, 16
NW = NC * NS
CH = TOT // NW
NVEC = CH // 16
NBATCH = CH // 128

_I32 = jnp.int32
_F32 = jnp.float32

def _wid():
    return lax.axis_index("s") * NC + lax.axis_index("c")


def _k1_body(groups_hbm, posw_hbm, prevw_hbm, cnt_hbm, lo_hbm,
             g_buf, posw_buf, prevw_buf, cnt_tbl, lo_tbl):
    wid = _wid()
    cbase = wid * CH
    pltpu.sync_copy(groups_hbm.at[pl.ds(cbase, CH)], g_buf)
    iota = lax.iota(_I32, 16)
    zeros16 = iota * 0
    false16 = iota < 0

    def init(i, _):
        cnt_tbl[pl.ds(i * 16, 16)] = zeros16
        lo_tbl[pl.ds(i * 16, 16)] = zeros16 - 1
        return 0

    lax.fori_loop(0, G // 16, init, 0)

    def step(v, _):
        off = v * 16
        g = g_buf[pl.ds(off, 16)]
        jvec = cbase + off + iota
        prior = zeros16
        prevj = zeros16
        found = false16
        anyf = false16
        for s in range(1, 16):
            gb = plsc.load_gather(g_buf, [off + jnp.maximum(iota - s, 0)])
            eqb = (gb == g) & (iota >= s)
            prevj = jnp.where(eqb & (~found), jvec - s, prevj)
            found = found | eqb
            prior = prior + eqb.astype(_I32)
            gf = plsc.load_gather(g_buf, [off + jnp.minimum(iota + s, 15)])
            anyf = anyf | ((gf == g) & (iota < 16 - s))
        cnt_cur = plsc.load_gather(cnt_tbl, [g])
        lo_cur = plsc.load_gather(lo_tbl, [g])
        posw = cnt_cur + prior
        prevv = jnp.where(found, prevj, lo_cur)
        islast = jnp.logical_not(anyf)
        plsc.store_scatter(cnt_tbl, [g], posw + 1, mask=islast)
        plsc.store_scatter(lo_tbl, [g], jvec, mask=islast)
        posw_buf[pl.ds(off, 16)] = posw
        prevw_buf[pl.ds(off, 16)] = prevv
        return 0

    lax.fori_loop(0, NVEC, step, 0)
    pltpu.sync_copy(posw_buf, posw_hbm.at[pl.ds(cbase, CH)])
    pltpu.sync_copy(prevw_buf, prevw_hbm.at[pl.ds(cbase, CH)])
    pltpu.sync_copy(cnt_tbl, cnt_hbm.at[wid])
    pltpu.sync_copy(lo_tbl, lo_hbm.at[wid])


_k1 = functools.partial(
    pl.kernel,
    out_type=(
        jax.ShapeDtypeStruct((TOT,), _I32),
        jax.ShapeDtypeStruct((TOT,), _I32),
        jax.ShapeDtypeStruct((NW, G), _I32),
        jax.ShapeDtypeStruct((NW, G), _I32),
    ),
    mesh=plsc.VectorSubcoreMesh(core_axis_name="c", subcore_axis_name="s"),
    compiler_params=pltpu.CompilerParams(needs_layout_passes=False),
    scratch_types=(
        pltpu.VMEM((CH,), _I32),
        pltpu.VMEM((CH,), _I32),
        pltpu.VMEM((CH,), _I32),
        pltpu.VMEM((G,), _I32),
        pltpu.VMEM((G,), _I32),
    ),
)(_k1_body)


def _k2_body(cnt_ref, lo_ref, base_ref, il_ref, hist_ref):
    cntf = cnt_ref[...].astype(_F32)
    lo = lo_ref[...]
    ri = lax.broadcasted_iota(_I32, (NW, NW), 0)
    ci = lax.broadcasted_iota(_I32, (NW, NW), 1)
    tril = (ci < ri).astype(_F32)
    pre = lax.dot(tril, cntf, preferred_element_type=_F32)
    hist = jnp.sum(cntf, axis=0, keepdims=True)
    gi = lax.broadcasted_iota(_I32, (G, G), 0)
    gj = lax.broadcasted_iota(_I32, (G, G), 1)
    sl = (gi < gj).astype(_F32)
    off = lax.dot(hist, sl, preferred_element_type=_F32)
    base_ref[...] = (off + pre).astype(_I32)
    m = lo
    for step in (1, 2, 4, 8, 16):
        shifted = jnp.concatenate(
            [jnp.full((step, G), -1, _I32), m[:-step]], axis=0)
        m = jnp.maximum(m, shifted)
    excl = jnp.concatenate([jnp.full((1, G), -1, _I32), m[:-1]], axis=0)
    gmax = m[-1:]
    il_ref[...] = jnp.where(excl >= 0, excl, gmax)
    hist_ref[...] = jnp.broadcast_to(hist.astype(_I32), (8, G))


def _k2(cnt, lo):
    return pl.pallas_call(
        _k2_body,
        out_shape=(
            jax.ShapeDtypeStruct((NW, G), _I32),
            jax.ShapeDtypeStruct((NW, G), _I32),
            jax.ShapeDtypeStruct((8, G), _I32),
        ),
    )(cnt, lo)


def _k3_body(emb_hbm, groups_hbm, neg_hbm, posw_hbm, prevw_hbm,
             base_hbm, il_hbm, hist_hbm,
             prow_hbm, nrow_hbm, w_hbm,
             g_buf, posw_b, prevw_b, base_t, il_t, hist_t,
             rank2d, part2d, negid2d, negg2d, w_buf,
             pos_v0, neg_v0, pos_v1, neg_v1, sem_g, sem_w):
    wid = _wid()
    cbase = wid * CH
    pltpu.sync_copy(groups_hbm.at[pl.ds(cbase, CH)], g_buf)
    pltpu.sync_copy(posw_hbm.at[pl.ds(cbase, CH)], posw_b)
    pltpu.sync_copy(prevw_hbm.at[pl.ds(cbase, CH)], prevw_b)
    pltpu.sync_copy(base_hbm.at[wid], base_t)
    pltpu.sync_copy(il_hbm.at[wid], il_t)
    pltpu.sync_copy(hist_hbm.at[0], hist_t)
    iota = lax.iota(_I32, 16)

    def phase_a(v, _):
        off = v * 16
        r = v // 8
        col = (v % 8) * 16
        g = g_buf[pl.ds(off, 16)]
        pw = posw_b[pl.ds(off, 16)]
        pv = prevw_b[pl.ds(off, 16)]
        rank = plsc.load_gather(base_t, [g]) + pw
        part = jnp.where(pv >= 0, pv, plsc.load_gather(il_t, [g]))
        rank2d[r, pl.ds(col, 16)] = rank
        part2d[r, pl.ds(col, 16)] = part
        return 0

    lax.fori_loop(0, NVEC, phase_a, 0)

    handles = [pltpu.async_copy(neg_hbm.at[rank2d.at[r]], negid2d.at[r], sem_g)
               for r in range(NBATCH)]
    for h in handles:
        h.wait()
    handles = [pltpu.async_copy(groups_hbm.at[negid2d.at[r]], negg2d.at[r],
                                sem_g)
               for r in range(NBATCH)]
    for h in handles:
        h.wait()

    def phase_w(v, _):
        off = v * 16
        g = g_buf[pl.ds(off, 16)]
        negg = negg2d[v // 8, pl.ds((v % 8) * 16, 16)]
        size = plsc.load_gather(hist_t, [g])
        w_buf[pl.ds(off, 16)] = ((size >= 2) & (negg != g)).astype(_F32)
        return 0

    lax.fori_loop(0, NVEC, phase_w, 0)
    pltpu.sync_copy(w_buf, w_hbm.at[pl.ds(cbase, CH)])

    bufs = ((pos_v0, neg_v0), (pos_v1, neg_v1))

    def gathers(r):
        pv, nv = bufs[r % 2]
        return (pltpu.async_copy(emb_hbm.at[part2d.at[r]], pv, sem_g),
                pltpu.async_copy(emb_hbm.at[negid2d.at[r]], nv, sem_g))

    gh = gathers(0)
    wh = None
    for r in range(NBATCH):
        gh[0].wait()
        gh[1].wait()
        if wh is not None:
            wh[0].wait()
            wh[1].wait()
        if r + 1 < NBATCH:
            gh = gathers(r + 1)
        pv, nv = bufs[r % 2]
        dst = pl.ds(cbase + r * 128, 128)
        wh = (pltpu.async_copy(pv, prow_hbm.at[dst], sem_w),
              pltpu.async_copy(nv, nrow_hbm.at[dst], sem_w))
    wh[0].wait()
    wh[1].wait()


_k3 = functools.partial(
    pl.kernel,
    out_type=(
        jax.ShapeDtypeStruct((TOT, 128), _F32),
        jax.ShapeDtypeStruct((TOT, 128), _F32),
        jax.ShapeDtypeStruct((TOT,), _F32),
    ),
    mesh=plsc.VectorSubcoreMesh(core_axis_name="c", subcore_axis_name="s"),
    compiler_params=pltpu.CompilerParams(needs_layout_passes=False),
    scratch_types=(
        pltpu.VMEM((CH,), _I32),
        pltpu.VMEM((CH,), _I32),
        pltpu.VMEM((CH,), _I32),
        pltpu.VMEM((G,), _I32),
        pltpu.VMEM((G,), _I32),
        pltpu.VMEM((G,), _I32),
        pltpu.VMEM((NBATCH, 128), _I32),
        pltpu.VMEM((NBATCH, 128), _I32),
        pltpu.VMEM((NBATCH, 128), _I32),
        pltpu.VMEM((NBATCH, 128), _I32),
        pltpu.VMEM((CH,), _F32),
        pltpu.VMEM((128, 128), _F32),
        pltpu.VMEM((128, 128), _F32),
        pltpu.VMEM((128, 128), _F32),
        pltpu.VMEM((128, 128), _F32),
        pltpu.SemaphoreType.DMA,
        pltpu.SemaphoreType.DMA,
    ),
)(_k3_body)


_K4_STEPS = 64
_K4_ROWS = TOT // _K4_STEPS


def _k4_body(emb_ref, pos_ref, neg_ref, w_ref, out_ref, num_acc, den_acc):
    i = pl.program_id(0)

    @pl.when(i == 0)
    def _():
        num_acc[...] = jnp.zeros((8, 128), _F32)
        den_acc[...] = jnp.zeros((8, 128), _F32)

    a = emb_ref[...]
    p = pos_ref[...][:, :64]
    q = neg_ref[...][:, :64]
    w = w_ref[...]
    ri = lax.broadcasted_iota(_I32, (128, 128), 0)
    ci = lax.broadcasted_iota(_I32, (128, 128), 1)
    eye = (ri == ci).astype(_F32)
    dn = (((1,), (1,)), ((), ()))
    nums = jnp.zeros((1, 128), _F32)
    dens = jnp.zeros((1, 128), _F32)
    for t in range(8):
        sl = slice(t * 128, (t + 1) * 128)
        at = a[sl]
        mp = lax.dot_general(at, p[sl], dn, preferred_element_type=_F32)
        mq = lax.dot_general(at, q[sl], dn, preferred_element_type=_F32)
        ps = jnp.sum(mp * eye, axis=0, keepdims=True)
        ns = jnp.sum(mq * eye, axis=0, keepdims=True)
        sc = (ns - ps) / TEMP
        sp = jnp.maximum(sc, 0.0) + jnp.log(1.0 + jnp.exp(-jnp.abs(sc)))
        wt = w[t:t + 1, :]
        nums = nums + sp * wt
        dens = dens + wt
    num_acc[0:1, :] = num_acc[0:1, :] + nums
    den_acc[0:1, :] = den_acc[0:1, :] + dens

    @pl.when(i == _K4_STEPS - 1)
    def _():
        num = jnp.sum(num_acc[...])
        den = jnp.sum(den_acc[...])
        out_ref[...] = jnp.full((8, 128), num / jnp.maximum(den, 1.0), _F32)


def _k4(emb, prow, nrow, w2):
    return pl.pallas_call(
        _k4_body,
        grid=(_K4_STEPS,),
        in_specs=[
            pl.BlockSpec((_K4_ROWS, 64), lambda i: (i, 0)),
            pl.BlockSpec((_K4_ROWS, 128), lambda i: (i, 0)),
            pl.BlockSpec((_K4_ROWS, 128), lambda i: (i, 0)),
            pl.BlockSpec((8, 128), lambda i: (i, 0)),
        ],
        out_specs=pl.BlockSpec((8, 128), lambda i: (0, 0)),
        out_shape=jax.ShapeDtypeStruct((8, 128), _F32),
        scratch_shapes=[pltpu.VMEM((8, 128), _F32),
                        pltpu.VMEM((8, 128), _F32)],
    )(emb, prow, nrow, w2)


def kernel(embeddings, groups):
    emb_flat = embeddings.reshape(TOT, D)
    emb_pad = jnp.pad(emb_flat, ((0, 0), (0, 128 - D)))
    g_flat = groups.reshape(TOT).astype(_I32)
    neg_const = jax.random.randint(jax.random.key(123), (TOT,), 0, TOT,
                                   dtype=_I32)
    posw, prevw, cnt, lo = _k1(g_flat)
    base, il, hist = _k2(cnt, lo)
    prow, nrow, w = _k3(emb_pad, g_flat, neg_const, posw, prevw, base, il,
                        hist)
    nd = _k4(emb_flat, prow, nrow, w.reshape(TOT // 128, 128))
    return nd[0, 0]


# speedup vs baseline: 1.8681x; 1.0553x over previous
import functools

import jax
import jax.numpy as jnp
import numpy as np
from jax import lax
from jax.experimental import pallas as pl
from jax.experimental.pallas import tpu as pltpu
from jax.experimental.pallas import tpu_sc as plsc

TEMP = 0.1
B, N, D = 16, 4096, 64
TOT = B * N
G = 1024
NC, NS = 2, 16
NW = NC * NS
CH = TOT // NW
NVEC = CH // 16
NBATCH = CH // 128

_I32 = jnp.int32
_F32 = jnp.float32

def _wid():
    return lax.axis_index("s") * NC + lax.axis_index("c")


def _k1_body(groups_hbm, posw_hbm, prevw_hbm, cnt_hbm, lo_hbm,
             g_buf, posw_buf, prevw_buf, cnt_tbl, lo_tbl):
    wid = _wid()
    cbase = wid * CH
    pltpu.sync_copy(groups_hbm.at[pl.ds(cbase, CH)], g_buf)
    iota = lax.iota(_I32, 16)
    zeros16 = iota * 0
    false16 = iota < 0

    def init(i, _):
        cnt_tbl[pl.ds(i * 16, 16)] = zeros16
        lo_tbl[pl.ds(i * 16, 16)] = zeros16 - 1
        return 0

    lax.fori_loop(0, G // 16, init, 0)

    def step(v, _):
        off = v * 16
        g = g_buf[pl.ds(off, 16)]
        jvec = cbase + off + iota
        prior = zeros16
        prevj = zeros16
        found = false16
        anyf = false16
        for s in range(1, 16):
            gb = plsc.load_gather(g_buf, [off + jnp.maximum(iota - s, 0)])
            eqb = (gb == g) & (iota >= s)
            prevj = jnp.where(eqb & (~found), jvec - s, prevj)
            found = found | eqb
            prior = prior + eqb.astype(_I32)
            gf = plsc.load_gather(g_buf, [off + jnp.minimum(iota + s, 15)])
            anyf = anyf | ((gf == g) & (iota < 16 - s))
        cnt_cur = plsc.load_gather(cnt_tbl, [g])
        lo_cur = plsc.load_gather(lo_tbl, [g])
        posw = cnt_cur + prior
        prevv = jnp.where(found, prevj, lo_cur)
        islast = jnp.logical_not(anyf)
        plsc.store_scatter(cnt_tbl, [g], posw + 1, mask=islast)
        plsc.store_scatter(lo_tbl, [g], jvec, mask=islast)
        posw_buf[pl.ds(off, 16)] = posw
        prevw_buf[pl.ds(off, 16)] = prevv
        return 0

    lax.fori_loop(0, NVEC, step, 0)
    pltpu.sync_copy(posw_buf, posw_hbm.at[pl.ds(cbase, CH)])
    pltpu.sync_copy(prevw_buf, prevw_hbm.at[pl.ds(cbase, CH)])
    pltpu.sync_copy(cnt_tbl, cnt_hbm.at[wid])
    pltpu.sync_copy(lo_tbl, lo_hbm.at[wid])


_k1 = functools.partial(
    pl.kernel,
    out_type=(
        jax.ShapeDtypeStruct((TOT,), _I32),
        jax.ShapeDtypeStruct((TOT,), _I32),
        jax.ShapeDtypeStruct((NW, G), _I32),
        jax.ShapeDtypeStruct((NW, G), _I32),
    ),
    mesh=plsc.VectorSubcoreMesh(core_axis_name="c", subcore_axis_name="s"),
    compiler_params=pltpu.CompilerParams(needs_layout_passes=False),
    scratch_types=(
        pltpu.VMEM((CH,), _I32),
        pltpu.VMEM((CH,), _I32),
        pltpu.VMEM((CH,), _I32),
        pltpu.VMEM((G,), _I32),
        pltpu.VMEM((G,), _I32),
    ),
)(_k1_body)


def _k2_body(cnt_ref, lo_ref, base_ref, il_ref, hist_ref):
    cntf = cnt_ref[...].astype(_F32)
    lo = lo_ref[...]
    ri = lax.broadcasted_iota(_I32, (NW, NW), 0)
    ci = lax.broadcasted_iota(_I32, (NW, NW), 1)
    tril = (ci < ri).astype(_F32)
    pre = lax.dot(tril, cntf, preferred_element_type=_F32)
    hist = jnp.sum(cntf, axis=0, keepdims=True)
    gi = lax.broadcasted_iota(_I32, (G, G), 0)
    gj = lax.broadcasted_iota(_I32, (G, G), 1)
    sl = (gi < gj).astype(_F32)
    off = lax.dot(hist, sl, preferred_element_type=_F32)
    base_ref[...] = (off + pre).astype(_I32)
    m = lo
    for step in (1, 2, 4, 8, 16):
        shifted = jnp.concatenate(
            [jnp.full((step, G), -1, _I32), m[:-step]], axis=0)
        m = jnp.maximum(m, shifted)
    excl = jnp.concatenate([jnp.full((1, G), -1, _I32), m[:-1]], axis=0)
    gmax = m[-1:]
    il_ref[...] = jnp.where(excl >= 0, excl, gmax)
    hist_ref[...] = jnp.broadcast_to(hist.astype(_I32), (8, G))


def _k2(cnt, lo):
    return pl.pallas_call(
        _k2_body,
        out_shape=(
            jax.ShapeDtypeStruct((NW, G), _I32),
            jax.ShapeDtypeStruct((NW, G), _I32),
            jax.ShapeDtypeStruct((8, G), _I32),
        ),
    )(cnt, lo)


def _k3_body(emb_hbm, groups_hbm, neg_hbm, posw_hbm, prevw_hbm,
             base_hbm, il_hbm, hist_hbm,
             prow_hbm, nrow_hbm, w_hbm,
             g_buf, posw_b, prevw_b, base_t, il_t, hist_t,
             rank2d, part2d, negid2d, negg2d, w_buf,
             pos_v0, neg_v0, pos_v1, neg_v1, sem_g, sem_w):
    wid = _wid()
    cbase = wid * CH
    pltpu.sync_copy(groups_hbm.at[pl.ds(cbase, CH)], g_buf)
    pltpu.sync_copy(posw_hbm.at[pl.ds(cbase, CH)], posw_b)
    pltpu.sync_copy(prevw_hbm.at[pl.ds(cbase, CH)], prevw_b)
    pltpu.sync_copy(base_hbm.at[wid], base_t)
    pltpu.sync_copy(il_hbm.at[wid], il_t)
    pltpu.sync_copy(hist_hbm.at[0], hist_t)
    iota = lax.iota(_I32, 16)

    def phase_a(v, _):
        off = v * 16
        r = v // 8
        col = (v % 8) * 16
        g = g_buf[pl.ds(off, 16)]
        pw = posw_b[pl.ds(off, 16)]
        pv = prevw_b[pl.ds(off, 16)]
        rank = plsc.load_gather(base_t, [g]) + pw
        part = jnp.where(pv >= 0, pv, plsc.load_gather(il_t, [g]))
        rank2d[r, pl.ds(col, 16)] = rank
        part2d[r, pl.ds(col, 16)] = part
        return 0

    lax.fori_loop(0, NVEC, phase_a, 0)

    handles = [pltpu.async_copy(neg_hbm.at[rank2d.at[r]], negid2d.at[r], sem_g)
               for r in range(NBATCH)]
    for h in handles:
        h.wait()
    handles = [pltpu.async_copy(groups_hbm.at[negid2d.at[r]], negg2d.at[r],
                                sem_g)
               for r in range(NBATCH)]
    for h in handles:
        h.wait()

    def phase_w(v, _):
        off = v * 16
        g = g_buf[pl.ds(off, 16)]
        negg = negg2d[v // 8, pl.ds((v % 8) * 16, 16)]
        size = plsc.load_gather(hist_t, [g])
        w_buf[pl.ds(off, 16)] = ((size >= 2) & (negg != g)).astype(_F32)
        return 0

    lax.fori_loop(0, NVEC, phase_w, 0)
    pltpu.sync_copy(w_buf, w_hbm.at[pl.ds(cbase, CH)])

    bufs = ((pos_v0, neg_v0), (pos_v1, neg_v1))

    def gathers(r):
        pv, nv = bufs[r % 2]
        return (pltpu.async_copy(emb_hbm.at[part2d.at[r]], pv, sem_g),
                pltpu.async_copy(emb_hbm.at[negid2d.at[r]], nv, sem_g))

    gh = gathers(0)
    wh = None
    for r in range(NBATCH):
        gh[0].wait()
        gh[1].wait()
        if wh is not None:
            wh[0].wait()
            wh[1].wait()
        if r + 1 < NBATCH:
            gh = gathers(r + 1)
        pv, nv = bufs[r % 2]
        dst = pl.ds(cbase + r * 128, 128)
        wh = (pltpu.async_copy(pv, prow_hbm.at[dst], sem_w),
              pltpu.async_copy(nv, nrow_hbm.at[dst], sem_w))
    wh[0].wait()
    wh[1].wait()


_k3 = functools.partial(
    pl.kernel,
    out_type=(
        jax.ShapeDtypeStruct((TOT, 128), _F32),
        jax.ShapeDtypeStruct((TOT, 128), _F32),
        jax.ShapeDtypeStruct((TOT,), _F32),
    ),
    mesh=plsc.VectorSubcoreMesh(core_axis_name="c", subcore_axis_name="s"),
    compiler_params=pltpu.CompilerParams(needs_layout_passes=False),
    scratch_types=(
        pltpu.VMEM((CH,), _I32),
        pltpu.VMEM((CH,), _I32),
        pltpu.VMEM((CH,), _I32),
        pltpu.VMEM((G,), _I32),
        pltpu.VMEM((G,), _I32),
        pltpu.VMEM((G,), _I32),
        pltpu.VMEM((NBATCH, 128), _I32),
        pltpu.VMEM((NBATCH, 128), _I32),
        pltpu.VMEM((NBATCH, 128), _I32),
        pltpu.VMEM((NBATCH, 128), _I32),
        pltpu.VMEM((CH,), _F32),
        pltpu.VMEM((128, 128), _F32),
        pltpu.VMEM((128, 128), _F32),
        pltpu.VMEM((128, 128), _F32),
        pltpu.VMEM((128, 128), _F32),
        pltpu.SemaphoreType.DMA,
        pltpu.SemaphoreType.DMA,
    ),
)(_k3_body)


_K4_STEPS = 64
_K4_ROWS = TOT // _K4_STEPS


def _k4_body(emb_ref, pos_ref, neg_ref, w_ref, out_ref, num_acc, den_acc):
    i = pl.program_id(0)

    @pl.when(i == 0)
    def _():
        num_acc[...] = jnp.zeros((8, 128), _F32)
        den_acc[...] = jnp.zeros((8, 128), _F32)

    a = emb_ref[...]
    p = pos_ref[...][:, :64]
    q = neg_ref[...][:, :64]
    w = w_ref[...]
    x = (q - p) * a
    ones8 = jnp.ones((8, 64), _F32)
    sd = lax.dot_general(ones8, x, (((1,), (1,)), ((), ())),
                         preferred_element_type=_F32)
    sc = sd[0:1, :] / TEMP
    sp = jnp.maximum(sc, 0.0) + jnp.log(1.0 + jnp.exp(-jnp.abs(sc)))
    sp8 = jnp.concatenate(
        [sp[:, t * 128:(t + 1) * 128] for t in range(8)], axis=0)
    num_acc[...] = num_acc[...] + sp8 * w
    den_acc[...] = den_acc[...] + w

    @pl.when(i == _K4_STEPS - 1)
    def _():
        num = jnp.sum(num_acc[...])
        den = jnp.sum(den_acc[...])
        out_ref[...] = jnp.full((8, 128), num / jnp.maximum(den, 1.0), _F32)


def _k4(emb, prow, nrow, w2):
    return pl.pallas_call(
        _k4_body,
        grid=(_K4_STEPS,),
        in_specs=[
            pl.BlockSpec((_K4_ROWS, 64), lambda i: (i, 0)),
            pl.BlockSpec((_K4_ROWS, 128), lambda i: (i, 0)),
            pl.BlockSpec((_K4_ROWS, 128), lambda i: (i, 0)),
            pl.BlockSpec((8, 128), lambda i: (i, 0)),
        ],
        out_specs=pl.BlockSpec((8, 128), lambda i: (0, 0)),
        out_shape=jax.ShapeDtypeStruct((8, 128), _F32),
        scratch_shapes=[pltpu.VMEM((8, 128), _F32),
                        pltpu.VMEM((8, 128), _F32)],
    )(emb, prow, nrow, w2)


def kernel(embeddings, groups):
    emb_flat = embeddings.reshape(TOT, D)
    emb_pad = jnp.pad(emb_flat, ((0, 0), (0, 128 - D)))
    g_flat = groups.reshape(TOT).astype(_I32)
    neg_const = jax.random.randint(jax.random.key(123), (TOT,), 0, TOT,
                                   dtype=_I32)
    posw, prevw, cnt, lo = _k1(g_flat)
    base, il, hist = _k2(cnt, lo)
    prow, nrow, w = _k3(emb_pad, g_flat, neg_const, posw, prevw, base, il,
                        hist)
    nd = _k4(emb_flat, prow, nrow, w.reshape(TOT // 128, 128))
    return nd[0, 0]
